# Initial kernel scaffold; baseline (speedup 1.0000x reference)
import jax
import jax.numpy as jnp
from jax.experimental import pallas as pl


def kernel(position_ids, maps):
    raise NotImplementedError("write your pallas kernel here")


# speedup vs baseline: 1.0758x; 1.0758x over previous
import functools

import jax
import jax.numpy as jnp
from jax import lax
from jax.experimental import pallas as pl
from jax.experimental.pallas import tpu as pltpu
from jax.experimental.pallas import tpu_sc as plsc

_DIM = 64
_ROW = _DIM * _DIM
_NC = 2
_NS = 16
_NW = _NC * _NS
_CHUNK = 8


@functools.lru_cache(maxsize=None)
def _make_gather(n, vocab):
    assert n % _NW == 0
    per_w = n // _NW
    assert per_w % _CHUNK == 0
    n_chunks = per_w // _CHUNK
    mesh = plsc.VectorSubcoreMesh(core_axis_name="c", subcore_axis_name="s")

    @functools.partial(
        pl.kernel,
        out_type=jax.ShapeDtypeStruct((n, _ROW), jnp.float32),
        mesh=mesh,
        scratch_types=[
            pltpu.VMEM((per_w,), jnp.int32),
            pltpu.VMEM((_CHUNK, _ROW), jnp.float32),
            pltpu.SemaphoreType.DMA,
        ],
    )
    def gather(table_hbm, idx_hbm, out_hbm, idx_v, rows_v, gsem):
        wid = lax.axis_index("s") * _NC + lax.axis_index("c")
        base = wid * per_w
        pltpu.sync_copy(idx_hbm.at[pl.ds(base, per_w)], idx_v)

        def body(c, carry):
            off = c * _CHUNK
            pltpu.async_copy(
                table_hbm.at[idx_v.at[pl.ds(off, _CHUNK)]], rows_v, gsem
            ).wait()
            pltpu.sync_copy(rows_v, out_hbm.at[pl.ds(base + off, _CHUNK)])
            return carry

        lax.fori_loop(0, n_chunks, body, 0)

    return gather


def kernel(position_ids, maps):
    b, s = position_ids.shape
    vocab = maps.shape[0]
    n = b * s
    table = maps.reshape(vocab, _ROW)
    idx = position_ids.reshape(n)
    out = _make_gather(n, vocab)(table, idx)
    return out.reshape(b, s, _DIM, _DIM)


# speedup vs baseline: 1.1228x; 1.0437x over previous
import functools

import jax
import jax.numpy as jnp
from jax import lax
from jax.experimental import pallas as pl
from jax.experimental.pallas import tpu as pltpu
from jax.experimental.pallas import tpu_sc as plsc

_DIM = 64
_ROW = _DIM * _DIM
_NC = 2
_NS = 16
_NW = _NC * _NS
_CHUNK = 8


_NBUF = 2


@functools.lru_cache(maxsize=None)
def _make_gather(n, vocab):
    assert n % _NW == 0
    per_w = n // _NW
    assert per_w % (_CHUNK * _NBUF) == 0
    n_chunks = per_w // _CHUNK
    n_rounds = n_chunks // _NBUF
    mesh = plsc.VectorSubcoreMesh(core_axis_name="c", subcore_axis_name="s")

    @functools.partial(
        pl.kernel,
        out_type=jax.ShapeDtypeStruct((n, _ROW), jnp.float32),
        mesh=mesh,
        scratch_types=[
            pltpu.VMEM((per_w,), jnp.int32),
            [pltpu.VMEM((_CHUNK, _ROW), jnp.float32) for _ in range(_NBUF)],
            [pltpu.SemaphoreType.DMA for _ in range(_NBUF)],
            [pltpu.SemaphoreType.DMA for _ in range(_NBUF)],
        ],
    )
    def gather(table_hbm, idx_hbm, out_hbm, idx_v, bufs, gsems, osems):
        wid = lax.axis_index("s") * _NC + lax.axis_index("c")
        base = wid * per_w
        pltpu.sync_copy(idx_hbm.at[pl.ds(base, per_w)], idx_v)

        def start_gather(c, b):
            pltpu.async_copy(
                table_hbm.at[idx_v.at[pl.ds(c * _CHUNK, _CHUNK)]],
                bufs[b],
                gsems[b],
            )

        def wait_gather(b):
            pltpu.make_async_copy(
                table_hbm.at[idx_v.at[pl.ds(0, _CHUNK)]], bufs[b], gsems[b]
            ).wait()

        def start_out(c, b):
            pltpu.async_copy(
                bufs[b], out_hbm.at[pl.ds(base + c * _CHUNK, _CHUNK)], osems[b]
            )

        def wait_out(b):
            pltpu.make_async_copy(
                bufs[b], out_hbm.at[pl.ds(base, _CHUNK)], osems[b]
            ).wait()

        for b in range(_NBUF):
            start_gather(b, b)

        def round_body(r, carry):
            for b in range(_NBUF):
                c = r * _NBUF + b
                wait_gather(b)
                start_out(c, b)
                @pl.when(c + _NBUF < n_chunks)
                def _():
                    wait_out(b)
                    start_gather(c + _NBUF, b)

            return carry

        lax.fori_loop(0, n_rounds, round_body, 0)
        for b in range(_NBUF):
            wait_out(b)

    return gather


def kernel(position_ids, maps):
    b, s = position_ids.shape
    vocab = maps.shape[0]
    n = b * s
    table = maps.reshape(vocab, _ROW)
    idx = position_ids.reshape(n)
    out = _make_gather(n, vocab)(table, idx)
    return out.reshape(b, s, _DIM, _DIM)
